# Initial kernel scaffold; baseline (speedup 1.0000x reference)
import jax
import jax.numpy as jnp
from jax.experimental import pallas as pl


def kernel(x, edge_index, edge_attr, timestamps, ln1_w, ln1_b, W, att_src, att_dst, gat_bias, ln2_w, ln2_b):
    raise NotImplementedError("write your pallas kernel here")


# speedup vs baseline: 37.2035x; 37.2035x over previous
import functools
import jax
import jax.numpy as jnp
from jax import lax
from jax.experimental import pallas as pl
from jax.experimental.pallas import tpu as pltpu
from jax.experimental.pallas import tpu_sc as plsc

N = 10000
D = 128
H = 4
C = 32
E = 320000

NC = 2
NS = 16
NW = NC * NS
EPW = E // NW
CHUNK = 64
SB = 16
NCH = 160
NSB = NCH // SB
EPW_PAD = NCH * CHUNK
EPAD = NW * EPW_PAD
NACC = N + 112
ACC_W = 144
RPT = NACC // NS
BLK = 1000


def _tc_pre(x, W, am, ln1w, ln1b):
    def body(x_ref, w_ref, am_ref, g_ref, b_ref, xt_ref, h_ref, as_ref, ad_ref):
        xv = x_ref[...]
        m = jnp.mean(xv, axis=1, keepdims=True)
        xc = xv - m
        v = jnp.mean(xc * xc, axis=1, keepdims=True)
        xt = xc * lax.rsqrt(v + 1e-5) * g_ref[...] + b_ref[...]
        xt_ref[...] = xt
        h = jnp.dot(xt, w_ref[...], preferred_element_type=jnp.float32)
        h_ref[...] = h
        a = jnp.dot(h, am_ref[...], preferred_element_type=jnp.float32)
        as_ref[...] = a[:, :H]
        ad_ref[...] = a[:, H:]

    grid = (N // BLK,)
    return pl.pallas_call(
        body,
        grid=grid,
        in_specs=[
            pl.BlockSpec((BLK, D), lambda i: (i, 0)),
            pl.BlockSpec((D, D), lambda i: (0, 0)),
            pl.BlockSpec((D, 2 * H), lambda i: (0, 0)),
            pl.BlockSpec((D,), lambda i: (0,)),
            pl.BlockSpec((D,), lambda i: (0,)),
        ],
        out_specs=[
            pl.BlockSpec((BLK, D), lambda i: (i, 0)),
            pl.BlockSpec((BLK, D), lambda i: (i, 0)),
            pl.BlockSpec((BLK, H), lambda i: (i, 0)),
            pl.BlockSpec((BLK, H), lambda i: (i, 0)),
        ],
        out_shape=[
            jax.ShapeDtypeStruct((N, D), jnp.float32),
            jax.ShapeDtypeStruct((N, D), jnp.float32),
            jax.ShapeDtypeStruct((N, H), jnp.float32),
            jax.ShapeDtypeStruct((N, H), jnp.float32),
        ],
    )(x, W, am, ln1w, ln1b)


def _sc_edge_pass(h, apair, si, di):
    mesh = plsc.VectorSubcoreMesh(core_axis_name="c", subcore_axis_name="s")

    def body(h_hbm, ap_hbm, si_hbm, di_hbm, part_hbm,
             acc_s, ap_s, sblk, dblk, rows_v, scat_v, asg_v, adg_v, ex_v, sem):
        cid = lax.axis_index("c")
        sid = lax.axis_index("s")
        wid = sid * NC + cid

        zrow = jnp.zeros((16,), jnp.float32)
        for r in range(CHUNK):
            for v in range(ACC_W // 16):
                scat_v[r, pl.ds(v * 16, 16)] = zrow
        zbase = sid * RPT
        nfull = RPT // CHUNK
        for k in range(nfull):
            pltpu.sync_copy(scat_v, acc_s.at[pl.ds(zbase + k * CHUNK, CHUNK)])
        rem = RPT - nfull * CHUNK
        pltpu.sync_copy(scat_v.at[pl.ds(0, rem)],
                        acc_s.at[pl.ds(zbase + nfull * CHUNK, rem)])

        @pl.when(sid == 0)
        def _():
            pltpu.sync_copy(ap_hbm, ap_s)

        plsc.subcore_barrier()

        li = lax.iota(jnp.int32, 16)

        def sb_body(sb, carry):
            rbase = wid * NCH + sb * SB
            pltpu.sync_copy(si_hbm.at[pl.ds(rbase, SB)], sblk)
            pltpu.sync_copy(di_hbm.at[pl.ds(rbase, SB)], dblk)

            def chunk_fn(c, carry2):
                cp = pltpu.async_copy(h_hbm.at[sblk.at[c]], rows_v, sem)
                pltpu.sync_copy(ap_s.at[sblk.at[c]], asg_v)
                pltpu.sync_copy(ap_s.at[dblk.at[c]], adg_v)

                def ex_group(g, carry3):
                    off = g * 16
                    e16 = li + off
                    for hh in range(H):
                        av = plsc.load_gather(
                            asg_v, [e16, jnp.full((16,), hh, jnp.int32)])
                        bv = plsc.load_gather(
                            adg_v, [e16, jnp.full((16,), hh + H, jnp.int32)])
                        z = av + bv
                        z = jnp.where(z >= 0.0, z, z * 0.2)
                        ex_v[hh, pl.ds(off, 16)] = jnp.exp(z)
                    return carry3

                lax.fori_loop(0, CHUNK // 16, ex_group, 0)
                cp.wait()

                def scale_group(g, carry3):
                    off = g * 16
                    e0 = ex_v[0, pl.ds(off, 16)]
                    e1 = ex_v[1, pl.ds(off, 16)]
                    e2 = ex_v[2, pl.ds(off, 16)]
                    e3 = ex_v[3, pl.ds(off, 16)]
                    for l in range(16):
                        e = off + l
                        s0 = jnp.full((16,), e0[l])
                        s1 = jnp.full((16,), e1[l])
                        s2 = jnp.full((16,), e2[l])
                        s3 = jnp.full((16,), e3[l])
                        sel = jnp.where(li == 0, s0, 0.0)
                        sel = jnp.where(li == 1, s1, sel)
                        sel = jnp.where(li == 2, s2, sel)
                        sel = jnp.where(li == 3, s3, sel)
                        scat_v[e, pl.ds(8 * 16, 16)] = sel
                        sc4 = (s0, s0, s1, s1, s2, s2, s3, s3)
                        for v in range(8):
                            scat_v[e, pl.ds(v * 16, 16)] = (
                                rows_v[e, pl.ds(v * 16, 16)] * sc4[v])
                    return carry3

                lax.fori_loop(0, CHUNK // 16, scale_group, 0)

                pltpu.sync_copy(scat_v, acc_s.at[dblk.at[c]], add=True)
                return carry2

            lax.fori_loop(0, SB, chunk_fn, 0)
            return carry

        lax.fori_loop(0, NSB, sb_body, 0)
        plsc.subcore_barrier()

        pltpu.sync_copy(acc_s.at[pl.ds(zbase, RPT)],
                        part_hbm.at[cid, pl.ds(zbase, RPT)])

    f = pl.kernel(
        body,
        out_type=jax.ShapeDtypeStruct((NC, NACC, ACC_W), jnp.float32),
        mesh=mesh,
        compiler_params=pltpu.CompilerParams(
            use_tc_tiling_on_sc=False, needs_layout_passes=False),
        scratch_types=[
            pltpu.VMEM_SHARED((NACC, ACC_W), jnp.float32),
            pltpu.VMEM_SHARED((NACC, 2 * H), jnp.float32),
            pltpu.VMEM((SB, CHUNK), jnp.int32),
            pltpu.VMEM((SB, CHUNK), jnp.int32),
            pltpu.VMEM((CHUNK, D), jnp.float32),
            pltpu.VMEM((CHUNK, ACC_W), jnp.float32),
            pltpu.VMEM((CHUNK, 2 * H), jnp.float32),
            pltpu.VMEM((CHUNK, 2 * H), jnp.float32),
            pltpu.VMEM((H, CHUNK), jnp.float32),
            pltpu.SemaphoreType.DMA,
        ],
    )
    return f(h, apair, si, di)


def _tc_post(part, xt, h, a_s, a_d, em, pm, qm, gb, ln2w, ln2b):
    def body(p_ref, xt_ref, h_ref, as_ref, ad_ref, em_ref, pm_ref, qm_ref,
             gb_ref, g_ref, b_ref, out_ref):
        acc = p_ref[0] + p_ref[1]
        z = as_ref[...] + ad_ref[...]
        z = jnp.where(z >= 0.0, z, 0.2 * z)
        exs = jnp.exp(z)
        hv = h_ref[...]
        exw = jnp.dot(exs, em_ref[...], preferred_element_type=jnp.float32)
        num = jnp.dot(acc, pm_ref[...], preferred_element_type=jnp.float32)
        num = num + exw * hv
        den = jnp.dot(acc, qm_ref[...], preferred_element_type=jnp.float32)
        den = den + exw
        xg = num / (den + 1e-16) + gb_ref[...]
        y = xt_ref[...] + xg
        m = jnp.mean(y, axis=1, keepdims=True)
        yc = y - m
        v = jnp.mean(yc * yc, axis=1, keepdims=True)
        out_ref[...] = yc * lax.rsqrt(v + 1e-5) * g_ref[...] + b_ref[...]

    grid = (N // BLK,)
    return pl.pallas_call(
        body,
        grid=grid,
        in_specs=[
            pl.BlockSpec((NC, BLK, ACC_W), lambda i: (0, i, 0)),
            pl.BlockSpec((BLK, D), lambda i: (i, 0)),
            pl.BlockSpec((BLK, D), lambda i: (i, 0)),
            pl.BlockSpec((BLK, H), lambda i: (i, 0)),
            pl.BlockSpec((BLK, H), lambda i: (i, 0)),
            pl.BlockSpec((H, D), lambda i: (0, 0)),
            pl.BlockSpec((ACC_W, D), lambda i: (0, 0)),
            pl.BlockSpec((ACC_W, D), lambda i: (0, 0)),
            pl.BlockSpec((D,), lambda i: (0,)),
            pl.BlockSpec((D,), lambda i: (0,)),
            pl.BlockSpec((D,), lambda i: (0,)),
        ],
        out_specs=pl.BlockSpec((BLK, D), lambda i: (i, 0)),
        out_shape=jax.ShapeDtypeStruct((N, D), jnp.float32),
    )(part, xt, h, a_s, a_d, em, pm, qm, gb, ln2w, ln2b)


def kernel(x, edge_index, edge_attr, timestamps, ln1_w, ln1_b, W, att_src,
           att_dst, gat_bias, ln2_w, ln2_b):
    eyeH = jnp.eye(H, dtype=jnp.float32)
    am_s = (att_src[:, :, None] * eyeH[:, None, :]).reshape(D, H)
    am_d = (att_dst[:, :, None] * eyeH[:, None, :]).reshape(D, H)
    am = jnp.concatenate([am_s, am_d], axis=1)
    em = jnp.repeat(eyeH, C, axis=1).reshape(H, D)
    pm = jnp.concatenate(
        [jnp.eye(D, dtype=jnp.float32),
         jnp.zeros((ACC_W - D, D), jnp.float32)], axis=0)
    qm = jnp.concatenate(
        [jnp.zeros((D, D), jnp.float32), em,
         jnp.zeros((ACC_W - D - H, D), jnp.float32)], axis=0)

    xt, h, a_s, a_d = _tc_pre(x, W, am, ln1_w, ln1_b)

    src = jnp.concatenate(
        [edge_index[0], jnp.zeros((EPAD - E,), jnp.int32)]).reshape(
            NW * NCH, CHUNK)
    dst = jnp.concatenate(
        [edge_index[1], jnp.full((EPAD - E,), N, jnp.int32)]).reshape(
            NW * NCH, CHUNK)
    zpad = jnp.zeros((NACC - N, 2 * H), jnp.float32)
    apair = jnp.concatenate(
        [jnp.concatenate([a_s, a_d], axis=1), zpad], axis=0)

    part = _sc_edge_pass(h, apair, src, dst)

    return _tc_post(part, xt, h, a_s, a_d, em, pm, qm,
                    gat_bias, ln2_w, ln2_b)


# speedup vs baseline: 58.4672x; 1.5716x over previous
import functools
import jax
import jax.numpy as jnp
from jax import lax
from jax.experimental import pallas as pl
from jax.experimental.pallas import tpu as pltpu
from jax.experimental.pallas import tpu_sc as plsc

N = 10000
D = 128
H = 4
C = 32
E = 320000

NC = 2
NS = 16
NW = NC * NS
EPW = E // NW
CHUNK = 64
NCH = 160
EPW_PAD = NCH * CHUNK
EPAD = NW * EPW_PAD
NACC = N + 112
EW = 16
RPT = NACC // NS
SBE = 512
BLK = 1000

_SC_PARAMS = pltpu.CompilerParams(
    use_tc_tiling_on_sc=False, needs_layout_passes=False)


def _tc_pre(x, W, am, ln1w, ln1b):
    def body(x_ref, w_ref, am_ref, g_ref, b_ref, xt_ref, h_ref, as_ref, ad_ref):
        xv = x_ref[...]
        m = jnp.mean(xv, axis=1, keepdims=True)
        xc = xv - m
        v = jnp.mean(xc * xc, axis=1, keepdims=True)
        xt = xc * lax.rsqrt(v + 1e-5) * g_ref[...] + b_ref[...]
        xt_ref[...] = xt
        h = jnp.dot(xt, w_ref[...], preferred_element_type=jnp.float32)
        h_ref[...] = h
        a = jnp.dot(h, am_ref[...], preferred_element_type=jnp.float32)
        as_ref[...] = a[:, :H]
        ad_ref[...] = a[:, H:]

    grid = (N // BLK,)
    return pl.pallas_call(
        body,
        grid=grid,
        in_specs=[
            pl.BlockSpec((BLK, D), lambda i: (i, 0)),
            pl.BlockSpec((D, D), lambda i: (0, 0)),
            pl.BlockSpec((D, 2 * H), lambda i: (0, 0)),
            pl.BlockSpec((D,), lambda i: (0,)),
            pl.BlockSpec((D,), lambda i: (0,)),
        ],
        out_specs=[
            pl.BlockSpec((BLK, D), lambda i: (i, 0)),
            pl.BlockSpec((BLK, D), lambda i: (i, 0)),
            pl.BlockSpec((BLK, H), lambda i: (i, 0)),
            pl.BlockSpec((BLK, H), lambda i: (i, 0)),
        ],
        out_shape=[
            jax.ShapeDtypeStruct((N, D), jnp.float32),
            jax.ShapeDtypeStruct((N, D), jnp.float32),
            jax.ShapeDtypeStruct((N, H), jnp.float32),
            jax.ShapeDtypeStruct((N, H), jnp.float32),
        ],
    )(x, W, am, ln1w, ln1b)


def _sc_ex_pass(apair, si1, di1):
    mesh = plsc.VectorSubcoreMesh(core_axis_name="c", subcore_axis_name="s")

    def body(ap_hbm, si_hbm, di_hbm, ex_hbm, ap_v, si_v, di_v, st0, st1, sem):
        cid = lax.axis_index("c")
        sid = lax.axis_index("s")
        wid = sid * NC + cid
        ebase = wid * EPW_PAD

        pltpu.sync_copy(ap_hbm, ap_v)
        pltpu.sync_copy(si_hbm.at[pl.ds(ebase, EPW_PAD)], si_v)
        pltpu.sync_copy(di_hbm.at[pl.ds(ebase, EPW_PAD)], di_v)

        li = lax.iota(jnp.int32, 16)
        stages = (st0, st1)
        NSB = EPW_PAD // SBE

        def build(sb, st):
            def grp(g, carry):
                off = sb * SBE + g * 16
                s16 = si_v[pl.ds(off, 16)]
                d16 = di_v[pl.ds(off, 16)]
                exs = []
                for hh in range(H):
                    av = plsc.load_gather(
                        ap_v, [s16, jnp.full((16,), hh, jnp.int32)])
                    bv = plsc.load_gather(
                        ap_v, [d16, jnp.full((16,), hh + H, jnp.int32)])
                    z = av + bv
                    z = jnp.where(z >= 0.0, z, z * 0.2)
                    exs.append(jnp.exp(z))
                for l in range(16):
                    sel = jnp.where(li == 0, jnp.full((16,), exs[0][l]), 0.0)
                    sel = jnp.where(li == 1, jnp.full((16,), exs[1][l]), sel)
                    sel = jnp.where(li == 2, jnp.full((16,), exs[2][l]), sel)
                    sel = jnp.where(li == 3, jnp.full((16,), exs[3][l]), sel)
                    st[g * 16 + l, pl.ds(0, 16)] = sel
                return carry

            lax.fori_loop(0, SBE // 16, grp, 0)

        def it(i, carry):
            for b in range(2):
                sb = 2 * i + b
                st = stages[b]

                @pl.when(i >= 1)
                def _():
                    pltpu.make_async_copy(
                        st, ex_hbm.at[pl.ds(0, SBE)], sem.at[b]).wait()

                build(sb, st)
                pltpu.async_copy(
                    st, ex_hbm.at[pl.ds(ebase + sb * SBE, SBE)], sem.at[b])
            return carry

        lax.fori_loop(0, NSB // 2, it, 0)
        for b in range(2):
            pltpu.make_async_copy(
                stages[b], ex_hbm.at[pl.ds(0, SBE)], sem.at[b]).wait()

    f = pl.kernel(
        body,
        out_type=jax.ShapeDtypeStruct((EPAD, EW), jnp.float32),
        mesh=mesh,
        compiler_params=_SC_PARAMS,
        scratch_types=[
            pltpu.VMEM((NACC, 2 * H), jnp.float32),
            pltpu.VMEM((EPW_PAD,), jnp.int32),
            pltpu.VMEM((EPW_PAD,), jnp.int32),
            pltpu.VMEM((SBE, EW), jnp.float32),
            pltpu.VMEM((SBE, EW), jnp.float32),
            pltpu.SemaphoreType.DMA((2,)),
        ],
    )
    return f(apair, si1, di1)


def _sc_edge_pass(h, exrow, si, di):
    mesh = plsc.VectorSubcoreMesh(core_axis_name="c", subcore_axis_name="s")

    def body(h_hbm, ex_hbm, si_hbm, di_hbm, ph_hbm, pe_hbm,
             acc_h, acc_e,
             r0, r1, r2, r3, x0, x1, x2, x3, i0, i1, i2, i3,
             sc0, sc1, sem_g, sem_s):
        cid = lax.axis_index("c")
        sid = lax.axis_index("s")
        wid = sid * NC + cid
        rbase = wid * NCH
        ebase = wid * EPW_PAD

        rows = (r0, r1, r2, r3)
        exb = (x0, x1, x2, x3)
        idx = (i0, i1, i2, i3)
        scat = (sc0, sc1)

        zrow = jnp.zeros((16,), jnp.float32)
        for r in range(CHUNK):
            for v in range(D // 16):
                sc0[r, pl.ds(v * 16, 16)] = zrow
            x0[r, pl.ds(0, 16)] = zrow
        zbase = sid * RPT
        nfull = RPT // CHUNK
        for k in range(nfull):
            pltpu.sync_copy(sc0, acc_h.at[pl.ds(zbase + k * CHUNK, CHUNK)])
            pltpu.sync_copy(x0, acc_e.at[pl.ds(zbase + k * CHUNK, CHUNK)])
        rem = RPT - nfull * CHUNK
        pltpu.sync_copy(sc0.at[pl.ds(0, rem)],
                        acc_h.at[pl.ds(zbase + nfull * CHUNK, rem)])
        pltpu.sync_copy(x0.at[pl.ds(0, rem)],
                        acc_e.at[pl.ds(zbase + nfull * CHUNK, rem)])
        plsc.subcore_barrier()

        def issue_gather(s, c):
            pltpu.async_copy(h_hbm.at[idx[s].at[0]], rows[s], sem_g.at[s])
            pltpu.async_copy(ex_hbm.at[pl.ds(ebase + c * CHUNK, CHUNK)],
                             exb[s], sem_g.at[s])

        def wait_gather(s):
            pltpu.make_async_copy(
                h_hbm.at[pl.ds(0, CHUNK)], rows[s], sem_g.at[s]).wait()
            pltpu.make_async_copy(
                ex_hbm.at[pl.ds(0, CHUNK)], exb[s], sem_g.at[s]).wait()

        def load_idx(s, c):
            pltpu.sync_copy(si_hbm.at[rbase + c], idx[s].at[0])
            pltpu.sync_copy(di_hbm.at[rbase + c], idx[s].at[1])

        def issue_scatter(s):
            pltpu.async_copy(scat[s % 2], acc_h.at[idx[s].at[1]],
                             sem_s.at[s], add=True)
            pltpu.async_copy(exb[s], acc_e.at[idx[s].at[1]], sem_s.at[s],
                             add=True)

        def wait_scatter(s):
            pltpu.make_async_copy(
                scat[s % 2], acc_h.at[pl.ds(0, CHUNK)], sem_s.at[s]).wait()
            pltpu.make_async_copy(
                exb[s], acc_e.at[pl.ds(0, CHUNK)], sem_s.at[s]).wait()

        def scale(s):
            out = scat[s % 2]
            mhi = jnp.full((16,), -65536, jnp.int32)

            def grp(g, carry):
                for l in range(16):
                    e = g * 16 + l
                    ev = exb[s][e, pl.ds(0, 16)]
                    sc4 = tuple(jnp.full((16,), ev[hh]) for hh in range(H))
                    for gg in range(H):
                        u = rows[s][e, pl.ds(gg * 32, 32)]
                        w = plsc.bitcast(u, jnp.int32)
                        lo = plsc.bitcast(
                            jnp.left_shift(w, 16), jnp.float32)
                        hi = plsc.bitcast(
                            jnp.bitwise_and(w, mhi), jnp.float32)
                        out[e, pl.ds(gg * 32, 16)] = lo * sc4[gg]
                        out[e, pl.ds(gg * 32 + 16, 16)] = hi * sc4[gg]
                return carry
            lax.fori_loop(0, CHUNK // 16, grp, 0)

        load_idx(0, 0)
        load_idx(1, 1)
        issue_gather(0, 0)

        def it(i, carry):
            for cp in range(4):
                c = 4 * i + cp
                wait_gather(cp)

                if cp >= 2:
                    wait_scatter(cp - 2)
                else:
                    @pl.when(i >= 1)
                    def _():
                        wait_scatter(cp + 2)

                scale(cp)

                if cp <= 2:
                    issue_gather(cp + 1, c + 1)
                else:
                    @pl.when(i < NCH // 4 - 1)
                    def _():
                        issue_gather(0, c + 1)

                if cp <= 1:
                    load_idx(cp + 2, c + 2)
                else:
                    @pl.when(i < NCH // 4 - 1)
                    def _():
                        load_idx(cp - 2, c + 2)

                issue_scatter(cp)
            return carry

        lax.fori_loop(0, NCH // 4, it, 0)
        wait_scatter(2)
        wait_scatter(3)
        plsc.subcore_barrier()

        pltpu.sync_copy(acc_h.at[pl.ds(zbase, RPT)],
                        ph_hbm.at[cid, pl.ds(zbase, RPT)])
        pltpu.sync_copy(acc_e.at[pl.ds(zbase, RPT)],
                        pe_hbm.at[cid, pl.ds(zbase, RPT)])

    f = pl.kernel(
        body,
        out_type=[
            jax.ShapeDtypeStruct((NC, NACC, D), jnp.float32),
            jax.ShapeDtypeStruct((NC, NACC, EW), jnp.float32),
        ],
        mesh=mesh,
        compiler_params=_SC_PARAMS,
        scratch_types=(
            [pltpu.VMEM_SHARED((NACC, D), jnp.float32),
             pltpu.VMEM_SHARED((NACC, EW), jnp.float32)]
            + [pltpu.VMEM((CHUNK, D), jnp.bfloat16)] * 4
            + [pltpu.VMEM((CHUNK, EW), jnp.float32)] * 4
            + [pltpu.VMEM((2, CHUNK), jnp.int32)] * 4
            + [pltpu.VMEM((CHUNK, D), jnp.float32)] * 2
            + [pltpu.SemaphoreType.DMA((4,)), pltpu.SemaphoreType.DMA((4,))]
        ),
    )
    return f(h, exrow, si, di)


def _tc_post(ph, pe, xt, h, a_s, a_d, em16, gb, ln2w, ln2b):
    def body(ph_ref, pe_ref, xt_ref, h_ref, as_ref, ad_ref, em_ref,
             gb_ref, g_ref, b_ref, out_ref):
        acc_h = ph_ref[0] + ph_ref[1]
        acc_e = pe_ref[0] + pe_ref[1]
        z = as_ref[...] + ad_ref[...]
        z = jnp.where(z >= 0.0, z, 0.2 * z)
        exs = jnp.exp(z)
        hv = h_ref[...]
        ew = em_ref[...]
        exw = jnp.dot(exs, ew[:H], preferred_element_type=jnp.float32)
        num = acc_h + exw * hv
        den = jnp.dot(acc_e, ew, preferred_element_type=jnp.float32) + exw
        xg = num / (den + 1e-16) + gb_ref[...]
        y = xt_ref[...] + xg
        m = jnp.mean(y, axis=1, keepdims=True)
        yc = y - m
        v = jnp.mean(yc * yc, axis=1, keepdims=True)
        out_ref[...] = yc * lax.rsqrt(v + 1e-5) * g_ref[...] + b_ref[...]

    grid = (N // BLK,)
    return pl.pallas_call(
        body,
        grid=grid,
        in_specs=[
            pl.BlockSpec((NC, BLK, D), lambda i: (0, i, 0)),
            pl.BlockSpec((NC, BLK, EW), lambda i: (0, i, 0)),
            pl.BlockSpec((BLK, D), lambda i: (i, 0)),
            pl.BlockSpec((BLK, D), lambda i: (i, 0)),
            pl.BlockSpec((BLK, H), lambda i: (i, 0)),
            pl.BlockSpec((BLK, H), lambda i: (i, 0)),
            pl.BlockSpec((EW, D), lambda i: (0, 0)),
            pl.BlockSpec((D,), lambda i: (0,)),
            pl.BlockSpec((D,), lambda i: (0,)),
            pl.BlockSpec((D,), lambda i: (0,)),
        ],
        out_specs=pl.BlockSpec((BLK, D), lambda i: (i, 0)),
        out_shape=jax.ShapeDtypeStruct((N, D), jnp.float32),
    )(ph, pe, xt, h, a_s, a_d, em16, gb, ln2w, ln2b)


def kernel(x, edge_index, edge_attr, timestamps, ln1_w, ln1_b, W, att_src,
           att_dst, gat_bias, ln2_w, ln2_b):
    eyeH = jnp.eye(H, dtype=jnp.float32)
    am_s = (att_src[:, :, None] * eyeH[:, None, :]).reshape(D, H)
    am_d = (att_dst[:, :, None] * eyeH[:, None, :]).reshape(D, H)
    am = jnp.concatenate([am_s, am_d], axis=1)
    em16 = jnp.concatenate(
        [jnp.repeat(eyeH, C, axis=1).reshape(H, D),
         jnp.zeros((EW - H, D), jnp.float32)], axis=0)

    xt, h, a_s, a_d = _tc_pre(x, W, am, ln1_w, ln1_b)

    src = jnp.concatenate([edge_index[0], jnp.zeros((EPAD - E,), jnp.int32)])
    dst = jnp.concatenate(
        [edge_index[1],
         N + jnp.arange(EPAD - E, dtype=jnp.int32) % (NACC - N)])
    zpad = jnp.zeros((NACC - N, 2 * H), jnp.float32)
    apair = jnp.concatenate(
        [jnp.concatenate([a_s, a_d], axis=1), zpad], axis=0)

    h_sw = h.reshape(N, H, 2, 16).transpose(0, 1, 3, 2).reshape(N, D)
    h_sw = h_sw.astype(jnp.bfloat16)

    exrow = _sc_ex_pass(apair, src, dst)
    part_h, part_e = _sc_edge_pass(
        h_sw, exrow, src.reshape(NW * NCH, CHUNK), dst.reshape(NW * NCH, CHUNK))

    return _tc_post(part_h, part_e, xt, h, a_s, a_d, em16,
                    gat_bias, ln2_w, ln2_b)


# speedup vs baseline: 75.3224x; 1.2883x over previous
import functools
import jax
import jax.numpy as jnp
from jax import lax
from jax.experimental import pallas as pl
from jax.experimental.pallas import tpu as pltpu
from jax.experimental.pallas import tpu_sc as plsc

N = 10000
D = 128
H = 4
C = 32
E = 320000

NC = 2
NS = 16
NW = NC * NS
EPW = E // NW
CHUNK = 64
NCH = 160
EPW_PAD = NCH * CHUNK
EPAD = NW * EPW_PAD
NACC = N + 112
EW = 16
RPT = NACC // NS
SBE = 512
BLK = 1000

_SC_PARAMS = pltpu.CompilerParams(
    use_tc_tiling_on_sc=False, needs_layout_passes=False)


def _tc_pre(x, W, am, ln1w, ln1b):
    def body(x_ref, w_ref, am_ref, g_ref, b_ref, xt_ref, h_ref, as_ref, ad_ref):
        xv = x_ref[...]
        m = jnp.mean(xv, axis=1, keepdims=True)
        xc = xv - m
        v = jnp.mean(xc * xc, axis=1, keepdims=True)
        xt = xc * lax.rsqrt(v + 1e-5) * g_ref[...] + b_ref[...]
        xt_ref[...] = xt
        h = jnp.dot(xt, w_ref[...], preferred_element_type=jnp.float32)
        h_ref[...] = h
        a = jnp.dot(h, am_ref[...], preferred_element_type=jnp.float32)
        as_ref[...] = a[:, :H]
        ad_ref[...] = a[:, H:]

    grid = (N // BLK,)
    return pl.pallas_call(
        body,
        grid=grid,
        in_specs=[
            pl.BlockSpec((BLK, D), lambda i: (i, 0)),
            pl.BlockSpec((D, D), lambda i: (0, 0)),
            pl.BlockSpec((D, 2 * H), lambda i: (0, 0)),
            pl.BlockSpec((D,), lambda i: (0,)),
            pl.BlockSpec((D,), lambda i: (0,)),
        ],
        out_specs=[
            pl.BlockSpec((BLK, D), lambda i: (i, 0)),
            pl.BlockSpec((BLK, D), lambda i: (i, 0)),
            pl.BlockSpec((BLK, H), lambda i: (i, 0)),
            pl.BlockSpec((BLK, H), lambda i: (i, 0)),
        ],
        out_shape=[
            jax.ShapeDtypeStruct((N, D), jnp.float32),
            jax.ShapeDtypeStruct((N, D), jnp.float32),
            jax.ShapeDtypeStruct((N, H), jnp.float32),
            jax.ShapeDtypeStruct((N, H), jnp.float32),
        ],
    )(x, W, am, ln1w, ln1b)


def _sc_ex_pass(apair, si1, di1):
    mesh = plsc.VectorSubcoreMesh(core_axis_name="c", subcore_axis_name="s")

    def body(ap_hbm, si_hbm, di_hbm, ex_hbm, ap_v, si_v, di_v, st0, st1, sem):
        cid = lax.axis_index("c")
        sid = lax.axis_index("s")
        wid = sid * NC + cid
        ebase = wid * EPW_PAD

        pltpu.sync_copy(ap_hbm, ap_v)
        pltpu.sync_copy(si_hbm.at[pl.ds(ebase, EPW_PAD)], si_v)
        pltpu.sync_copy(di_hbm.at[pl.ds(ebase, EPW_PAD)], di_v)

        li = lax.iota(jnp.int32, 16)
        stages = (st0, st1)
        NSB = EPW_PAD // SBE

        def build(sb, st):
            def grp(g, carry):
                off = sb * SBE + g * 16
                s16 = si_v[pl.ds(off, 16)]
                d16 = di_v[pl.ds(off, 16)]
                exs = []
                for hh in range(H):
                    av = plsc.load_gather(
                        ap_v, [s16, jnp.full((16,), hh, jnp.int32)])
                    bv = plsc.load_gather(
                        ap_v, [d16, jnp.full((16,), hh + H, jnp.int32)])
                    z = av + bv
                    z = jnp.where(z >= 0.0, z, z * 0.2)
                    exs.append(jnp.exp(z))
                for l in range(16):
                    sel = jnp.where(li == 0, jnp.full((16,), exs[0][l]), 0.0)
                    sel = jnp.where(li == 1, jnp.full((16,), exs[1][l]), sel)
                    sel = jnp.where(li == 2, jnp.full((16,), exs[2][l]), sel)
                    sel = jnp.where(li == 3, jnp.full((16,), exs[3][l]), sel)
                    st[g * 16 + l, pl.ds(0, 16)] = sel
                return carry

            lax.fori_loop(0, SBE // 16, grp, 0)

        def it(i, carry):
            for b in range(2):
                sb = 2 * i + b
                st = stages[b]

                @pl.when(i >= 1)
                def _():
                    pltpu.make_async_copy(
                        st, ex_hbm.at[pl.ds(0, SBE)], sem.at[b]).wait()

                build(sb, st)
                pltpu.async_copy(
                    st, ex_hbm.at[pl.ds(ebase + sb * SBE, SBE)], sem.at[b])
            return carry

        lax.fori_loop(0, NSB // 2, it, 0)
        for b in range(2):
            pltpu.make_async_copy(
                stages[b], ex_hbm.at[pl.ds(0, SBE)], sem.at[b]).wait()

    f = pl.kernel(
        body,
        out_type=jax.ShapeDtypeStruct((EPAD, EW), jnp.float32),
        mesh=mesh,
        compiler_params=_SC_PARAMS,
        scratch_types=[
            pltpu.VMEM((NACC, 2 * H), jnp.float32),
            pltpu.VMEM((EPW_PAD,), jnp.int32),
            pltpu.VMEM((EPW_PAD,), jnp.int32),
            pltpu.VMEM((SBE, EW), jnp.float32),
            pltpu.VMEM((SBE, EW), jnp.float32),
            pltpu.SemaphoreType.DMA((2,)),
        ],
    )
    return f(apair, si1, di1)


def _sc_edge_pass(h, exrow, si, di):
    mesh = plsc.VectorSubcoreMesh(core_axis_name="c", subcore_axis_name="s")

    def body(h_hbm, ex_hbm, si_hbm, di_hbm, ph_hbm, pe_hbm,
             acc_h, acc_e,
             r0, r1, r2, r3, x0, x1, x2, x3, i0, i1, i2, i3,
             sc0, sc1, sem_g, sem_s):
        cid = lax.axis_index("c")
        sid = lax.axis_index("s")
        wid = sid * NC + cid
        rbase = wid * NCH
        ebase = wid * EPW_PAD

        rows = (r0, r1, r2, r3)
        exb = (x0, x1, x2, x3)
        idx = (i0, i1, i2, i3)
        scat = (sc0, sc1)

        zrow = jnp.zeros((16,), jnp.float32)
        for r in range(CHUNK):
            for v in range(D // 16):
                sc0[r, pl.ds(v * 16, 16)] = zrow
            x0[r, pl.ds(0, 16)] = zrow
        zbase = sid * RPT
        nfull = RPT // CHUNK
        for k in range(nfull):
            pltpu.sync_copy(sc0, acc_h.at[pl.ds(zbase + k * CHUNK, CHUNK)])
            pltpu.sync_copy(x0, acc_e.at[pl.ds(zbase + k * CHUNK, CHUNK)])
        rem = RPT - nfull * CHUNK
        pltpu.sync_copy(sc0.at[pl.ds(0, rem)],
                        acc_h.at[pl.ds(zbase + nfull * CHUNK, rem)])
        pltpu.sync_copy(x0.at[pl.ds(0, rem)],
                        acc_e.at[pl.ds(zbase + nfull * CHUNK, rem)])
        plsc.subcore_barrier()

        def issue_gather(s, c):
            pltpu.async_copy(h_hbm.at[idx[s].at[0]], rows[s], sem_g.at[s])
            pltpu.async_copy(ex_hbm.at[pl.ds(ebase + c * CHUNK, CHUNK)],
                             exb[s], sem_g.at[s])

        def wait_gather(s):
            pltpu.make_async_copy(
                h_hbm.at[pl.ds(0, CHUNK)], rows[s], sem_g.at[s]).wait()
            pltpu.make_async_copy(
                ex_hbm.at[pl.ds(0, CHUNK)], exb[s], sem_g.at[s]).wait()

        def load_idx(s, c):
            pltpu.sync_copy(si_hbm.at[rbase + c], idx[s].at[0])
            pltpu.sync_copy(di_hbm.at[rbase + c], idx[s].at[1])

        def issue_scatter(s):
            pltpu.async_copy(scat[s % 2], acc_h.at[idx[s].at[1]],
                             sem_s.at[s], add=True)
            pltpu.async_copy(exb[s], acc_e.at[idx[s].at[1]], sem_s.at[s],
                             add=True)

        def wait_scatter(s):
            pltpu.make_async_copy(
                scat[s % 2], acc_h.at[pl.ds(0, CHUNK)], sem_s.at[s]).wait()
            pltpu.make_async_copy(
                exb[s], acc_e.at[pl.ds(0, CHUNK)], sem_s.at[s]).wait()

        def scale(s):
            out = scat[s % 2]
            mhi = jnp.full((16,), -65536, jnp.int32)

            def grp(g, carry):
                for l in range(16):
                    e = g * 16 + l
                    ev = exb[s][e, pl.ds(0, 16)]
                    sc4 = tuple(jnp.full((16,), ev[hh]) for hh in range(H))
                    for gg in range(H):
                        u = rows[s][e, pl.ds(gg * 32, 32)]
                        w = plsc.bitcast(u, jnp.int32)
                        lo = plsc.bitcast(
                            jnp.left_shift(w, 16), jnp.float32)
                        hi = plsc.bitcast(
                            jnp.bitwise_and(w, mhi), jnp.float32)
                        out[e, pl.ds(gg * 32, 16)] = lo * sc4[gg]
                        out[e, pl.ds(gg * 32 + 16, 16)] = hi * sc4[gg]
                return carry
            lax.fori_loop(0, CHUNK // 16, grp, 0)

        load_idx(0, 0)
        load_idx(1, 1)
        issue_gather(0, 0)
        issue_gather(1, 1)

        def it(i, carry):
            for cp in range(4):
                c = 4 * i + cp
                wait_gather(cp)

                if cp >= 2:
                    wait_scatter(cp - 2)
                else:
                    @pl.when(i >= 1)
                    def _():
                        wait_scatter(cp + 2)

                if cp <= 1:
                    load_idx(cp + 2, c + 2)
                    issue_gather(cp + 2, c + 2)
                else:
                    @pl.when(i < NCH // 4 - 1)
                    def _():
                        load_idx(cp - 2, c + 2)
                        issue_gather(cp - 2, c + 2)

                scale(cp)
                issue_scatter(cp)
            return carry

        lax.fori_loop(0, NCH // 4, it, 0)
        wait_scatter(2)
        wait_scatter(3)
        plsc.subcore_barrier()

        pltpu.sync_copy(acc_h.at[pl.ds(zbase, RPT)],
                        ph_hbm.at[cid, pl.ds(zbase, RPT)])
        pltpu.sync_copy(acc_e.at[pl.ds(zbase, RPT)],
                        pe_hbm.at[cid, pl.ds(zbase, RPT)])

    f = pl.kernel(
        body,
        out_type=[
            jax.ShapeDtypeStruct((NC, NACC, D), jnp.float32),
            jax.ShapeDtypeStruct((NC, NACC, EW), jnp.float32),
        ],
        mesh=mesh,
        compiler_params=_SC_PARAMS,
        scratch_types=(
            [pltpu.VMEM_SHARED((NACC, D), jnp.float32),
             pltpu.VMEM_SHARED((NACC, EW), jnp.float32)]
            + [pltpu.VMEM((CHUNK, D), jnp.bfloat16)] * 4
            + [pltpu.VMEM((CHUNK, EW), jnp.float32)] * 4
            + [pltpu.VMEM((2, CHUNK), jnp.int32)] * 4
            + [pltpu.VMEM((CHUNK, D), jnp.float32)] * 2
            + [pltpu.SemaphoreType.DMA((4,)), pltpu.SemaphoreType.DMA((4,))]
        ),
    )
    return f(h, exrow, si, di)


def _tc_post(ph, pe, xt, h, a_s, a_d, em16, gb, ln2w, ln2b):
    def body(ph_ref, pe_ref, xt_ref, h_ref, as_ref, ad_ref, em_ref,
             gb_ref, g_ref, b_ref, out_ref):
        acc_h = ph_ref[0] + ph_ref[1]
        acc_e = pe_ref[0] + pe_ref[1]
        z = as_ref[...] + ad_ref[...]
        z = jnp.where(z >= 0.0, z, 0.2 * z)
        exs = jnp.exp(z)
        hv = h_ref[...]
        ew = em_ref[...]
        exw = jnp.dot(exs, ew[:H], preferred_element_type=jnp.float32)
        num = acc_h + exw * hv
        den = jnp.dot(acc_e, ew, preferred_element_type=jnp.float32) + exw
        xg = num / (den + 1e-16) + gb_ref[...]
        y = xt_ref[...] + xg
        m = jnp.mean(y, axis=1, keepdims=True)
        yc = y - m
        v = jnp.mean(yc * yc, axis=1, keepdims=True)
        out_ref[...] = yc * lax.rsqrt(v + 1e-5) * g_ref[...] + b_ref[...]

    grid = (N // BLK,)
    return pl.pallas_call(
        body,
        grid=grid,
        in_specs=[
            pl.BlockSpec((NC, BLK, D), lambda i: (0, i, 0)),
            pl.BlockSpec((NC, BLK, EW), lambda i: (0, i, 0)),
            pl.BlockSpec((BLK, D), lambda i: (i, 0)),
            pl.BlockSpec((BLK, D), lambda i: (i, 0)),
            pl.BlockSpec((BLK, H), lambda i: (i, 0)),
            pl.BlockSpec((BLK, H), lambda i: (i, 0)),
            pl.BlockSpec((EW, D), lambda i: (0, 0)),
            pl.BlockSpec((D,), lambda i: (0,)),
            pl.BlockSpec((D,), lambda i: (0,)),
            pl.BlockSpec((D,), lambda i: (0,)),
        ],
        out_specs=pl.BlockSpec((BLK, D), lambda i: (i, 0)),
        out_shape=jax.ShapeDtypeStruct((N, D), jnp.float32),
    )(ph, pe, xt, h, a_s, a_d, em16, gb, ln2w, ln2b)


def kernel(x, edge_index, edge_attr, timestamps, ln1_w, ln1_b, W, att_src,
           att_dst, gat_bias, ln2_w, ln2_b):
    eyeH = jnp.eye(H, dtype=jnp.float32)
    am_s = (att_src[:, :, None] * eyeH[:, None, :]).reshape(D, H)
    am_d = (att_dst[:, :, None] * eyeH[:, None, :]).reshape(D, H)
    am = jnp.concatenate([am_s, am_d], axis=1)
    em16 = jnp.concatenate(
        [jnp.repeat(eyeH, C, axis=1).reshape(H, D),
         jnp.zeros((EW - H, D), jnp.float32)], axis=0)

    xt, h, a_s, a_d = _tc_pre(x, W, am, ln1_w, ln1_b)

    src = jnp.concatenate([edge_index[0], jnp.zeros((EPAD - E,), jnp.int32)])
    dst = jnp.concatenate(
        [edge_index[1],
         N + jnp.arange(EPAD - E, dtype=jnp.int32) % (NACC - N)])
    zpad = jnp.zeros((NACC - N, 2 * H), jnp.float32)
    apair = jnp.concatenate(
        [jnp.concatenate([a_s, a_d], axis=1), zpad], axis=0)

    h_sw = h.reshape(N, H, 2, 16).transpose(0, 1, 3, 2).reshape(N, D)
    h_sw = h_sw.astype(jnp.bfloat16)

    exrow = _sc_ex_pass(apair, src, dst)
    part_h, part_e = _sc_edge_pass(
        h_sw, exrow, src.reshape(NW * NCH, CHUNK), dst.reshape(NW * NCH, CHUNK))

    return _tc_post(part_h, part_e, xt, h, a_s, a_d, em16,
                    gat_bias, ln2_w, ln2_b)


# speedup vs baseline: 76.9903x; 1.0221x over previous
import functools
import jax
import jax.numpy as jnp
from jax import lax
from jax.experimental import pallas as pl
from jax.experimental.pallas import tpu as pltpu
from jax.experimental.pallas import tpu_sc as plsc

N = 10000
D = 128
H = 4
C = 32
E = 320000

NC = 2
NS = 16
NW = NC * NS
EPW = E // NW
CHUNK = 64
NCH = 160
EPW_PAD = NCH * CHUNK
EPAD = NW * EPW_PAD
NACC = N + 112
EW = 16
ACC_W = D + EW
RPT = NACC // NS
SBE = 512
BLK = 1000

_SC_PARAMS = pltpu.CompilerParams(
    use_tc_tiling_on_sc=False, needs_layout_passes=False)


def _tc_pre(x, W, am, ln1w, ln1b):
    def body(x_ref, w_ref, am_ref, g_ref, b_ref, xt_ref, h_ref, as_ref, ad_ref):
        xv = x_ref[...]
        m = jnp.mean(xv, axis=1, keepdims=True)
        xc = xv - m
        v = jnp.mean(xc * xc, axis=1, keepdims=True)
        xt = xc * lax.rsqrt(v + 1e-5) * g_ref[...] + b_ref[...]
        xt_ref[...] = xt
        h = jnp.dot(xt, w_ref[...], preferred_element_type=jnp.float32)
        h_ref[...] = h
        a = jnp.dot(h, am_ref[...], preferred_element_type=jnp.float32)
        as_ref[...] = a[:, :H]
        ad_ref[...] = a[:, H:]

    grid = (N // BLK,)
    return pl.pallas_call(
        body,
        grid=grid,
        in_specs=[
            pl.BlockSpec((BLK, D), lambda i: (i, 0)),
            pl.BlockSpec((D, D), lambda i: (0, 0)),
            pl.BlockSpec((D, 2 * H), lambda i: (0, 0)),
            pl.BlockSpec((D,), lambda i: (0,)),
            pl.BlockSpec((D,), lambda i: (0,)),
        ],
        out_specs=[
            pl.BlockSpec((BLK, D), lambda i: (i, 0)),
            pl.BlockSpec((BLK, D), lambda i: (i, 0)),
            pl.BlockSpec((BLK, H), lambda i: (i, 0)),
            pl.BlockSpec((BLK, H), lambda i: (i, 0)),
        ],
        out_shape=[
            jax.ShapeDtypeStruct((N, D), jnp.float32),
            jax.ShapeDtypeStruct((N, D), jnp.float32),
            jax.ShapeDtypeStruct((N, H), jnp.float32),
            jax.ShapeDtypeStruct((N, H), jnp.float32),
        ],
    )(x, W, am, ln1w, ln1b)


def _sc_ex_pass(apair, si1, di1):
    mesh = plsc.VectorSubcoreMesh(core_axis_name="c", subcore_axis_name="s")

    def body(ap_hbm, si_hbm, di_hbm, ex_hbm, ap_v, si_v, di_v, st0, st1, sem):
        cid = lax.axis_index("c")
        sid = lax.axis_index("s")
        wid = sid * NC + cid
        ebase = wid * EPW_PAD

        pltpu.sync_copy(ap_hbm, ap_v)
        pltpu.sync_copy(si_hbm.at[pl.ds(ebase, EPW_PAD)], si_v)
        pltpu.sync_copy(di_hbm.at[pl.ds(ebase, EPW_PAD)], di_v)

        li = lax.iota(jnp.int32, 16)
        stages = (st0, st1)
        NSB = EPW_PAD // SBE

        def build(sb, st):
            def grp(g, carry):
                off = sb * SBE + g * 16
                s16 = si_v[pl.ds(off, 16)]
                d16 = di_v[pl.ds(off, 16)]
                exs = []
                for hh in range(H):
                    av = plsc.load_gather(
                        ap_v, [s16, jnp.full((16,), hh, jnp.int32)])
                    bv = plsc.load_gather(
                        ap_v, [d16, jnp.full((16,), hh + H, jnp.int32)])
                    z = av + bv
                    z = jnp.where(z >= 0.0, z, z * 0.2)
                    exs.append(jnp.exp(z))
                for l in range(16):
                    sel = jnp.where(li == 0, jnp.full((16,), exs[0][l]), 0.0)
                    sel = jnp.where(li == 1, jnp.full((16,), exs[1][l]), sel)
                    sel = jnp.where(li == 2, jnp.full((16,), exs[2][l]), sel)
                    sel = jnp.where(li == 3, jnp.full((16,), exs[3][l]), sel)
                    st[g * 16 + l, pl.ds(0, 16)] = sel
                return carry

            lax.fori_loop(0, SBE // 16, grp, 0)

        def it(i, carry):
            for b in range(2):
                sb = 2 * i + b
                st = stages[b]

                @pl.when(i >= 1)
                def _():
                    pltpu.make_async_copy(
                        st, ex_hbm.at[pl.ds(0, SBE)], sem.at[b]).wait()

                build(sb, st)
                pltpu.async_copy(
                    st, ex_hbm.at[pl.ds(ebase + sb * SBE, SBE)], sem.at[b])
            return carry

        lax.fori_loop(0, NSB // 2, it, 0)
        for b in range(2):
            pltpu.make_async_copy(
                stages[b], ex_hbm.at[pl.ds(0, SBE)], sem.at[b]).wait()

    f = pl.kernel(
        body,
        out_type=jax.ShapeDtypeStruct((EPAD, EW), jnp.float32),
        mesh=mesh,
        compiler_params=_SC_PARAMS,
        scratch_types=[
            pltpu.VMEM((NACC, 2 * H), jnp.float32),
            pltpu.VMEM((EPW_PAD,), jnp.int32),
            pltpu.VMEM((EPW_PAD,), jnp.int32),
            pltpu.VMEM((SBE, EW), jnp.float32),
            pltpu.VMEM((SBE, EW), jnp.float32),
            pltpu.SemaphoreType.DMA((2,)),
        ],
    )
    return f(apair, si1, di1)


def _sc_edge_pass(h, exrow, sdi):
    mesh = plsc.VectorSubcoreMesh(core_axis_name="c", subcore_axis_name="s")

    def body(h_hbm, ex_hbm, sdi_hbm, part_hbm,
             acc,
             r0, r1, r2, r3, x0, x1, x2, x3, i0, i1, i2, i3,
             sc0, sc1, sem_g, sem_s):
        cid = lax.axis_index("c")
        sid = lax.axis_index("s")
        wid = sid * NC + cid
        rbase = wid * NCH
        ebase = wid * EPW_PAD

        rows = (r0, r1, r2, r3)
        exb = (x0, x1, x2, x3)
        idx = (i0, i1, i2, i3)
        scat = (sc0, sc1)

        zrow = jnp.zeros((16,), jnp.float32)
        for r in range(CHUNK):
            for v in range(ACC_W // 16):
                sc0[r, pl.ds(v * 16, 16)] = zrow
        zbase = sid * RPT
        nfull = RPT // CHUNK
        for k in range(nfull):
            pltpu.sync_copy(sc0, acc.at[pl.ds(zbase + k * CHUNK, CHUNK)])
        rem = RPT - nfull * CHUNK
        pltpu.sync_copy(sc0.at[pl.ds(0, rem)],
                        acc.at[pl.ds(zbase + nfull * CHUNK, rem)])
        plsc.subcore_barrier()

        def issue_gather(s, c):
            pltpu.async_copy(h_hbm.at[idx[s].at[0]], rows[s], sem_g.at[s])
            pltpu.async_copy(ex_hbm.at[pl.ds(ebase + c * CHUNK, CHUNK)],
                             exb[s], sem_g.at[s])

        def wait_gather(s):
            pltpu.make_async_copy(
                h_hbm.at[pl.ds(0, CHUNK)], rows[s], sem_g.at[s]).wait()
            pltpu.make_async_copy(
                ex_hbm.at[pl.ds(0, CHUNK)], exb[s], sem_g.at[s]).wait()

        def load_idx(s, c):
            pltpu.sync_copy(sdi_hbm.at[rbase + c], idx[s])

        def issue_scatter(s):
            pltpu.async_copy(scat[s % 2], acc.at[idx[s].at[1]],
                             sem_s.at[s], add=True)

        def wait_scatter(s):
            pltpu.make_async_copy(
                scat[s % 2], acc.at[pl.ds(0, CHUNK)], sem_s.at[s]).wait()

        def scale(s):
            out = scat[s % 2]
            mhi = jnp.full((16,), -65536, jnp.int32)

            def grp(g, carry):
                for l in range(16):
                    e = g * 16 + l
                    ev = exb[s][e, pl.ds(0, 16)]
                    out[e, pl.ds(D, 16)] = ev
                    sc4 = tuple(jnp.full((16,), ev[hh]) for hh in range(H))
                    for gg in range(H):
                        u = rows[s][e, pl.ds(gg * 32, 32)]
                        w = plsc.bitcast(u, jnp.int32)
                        lo = plsc.bitcast(
                            jnp.left_shift(w, 16), jnp.float32)
                        hi = plsc.bitcast(
                            jnp.bitwise_and(w, mhi), jnp.float32)
                        out[e, pl.ds(gg * 32, 16)] = lo * sc4[gg]
                        out[e, pl.ds(gg * 32 + 16, 16)] = hi * sc4[gg]
                return carry
            lax.fori_loop(0, CHUNK // 16, grp, 0)

        load_idx(0, 0)
        load_idx(1, 1)
        issue_gather(0, 0)
        issue_gather(1, 1)

        def it(i, carry):
            for cp in range(4):
                c = 4 * i + cp
                wait_gather(cp)

                if cp >= 2:
                    wait_scatter(cp - 2)
                else:
                    @pl.when(i >= 1)
                    def _():
                        wait_scatter(cp + 2)

                if cp <= 1:
                    load_idx(cp + 2, c + 2)
                    issue_gather(cp + 2, c + 2)
                else:
                    @pl.when(i < NCH // 4 - 1)
                    def _():
                        load_idx(cp - 2, c + 2)
                        issue_gather(cp - 2, c + 2)

                scale(cp)
                issue_scatter(cp)
            return carry

        lax.fori_loop(0, NCH // 4, it, 0)
        wait_scatter(2)
        wait_scatter(3)
        plsc.subcore_barrier()

        pltpu.sync_copy(acc.at[pl.ds(zbase, RPT)],
                        part_hbm.at[cid, pl.ds(zbase, RPT)])

    f = pl.kernel(
        body,
        out_type=jax.ShapeDtypeStruct((NC, NACC, ACC_W), jnp.float32),
        mesh=mesh,
        compiler_params=_SC_PARAMS,
        scratch_types=(
            [pltpu.VMEM_SHARED((NACC, ACC_W), jnp.float32)]
            + [pltpu.VMEM((CHUNK, D), jnp.bfloat16)] * 4
            + [pltpu.VMEM((CHUNK, EW), jnp.float32)] * 4
            + [pltpu.VMEM((2, CHUNK), jnp.int32)] * 4
            + [pltpu.VMEM((CHUNK, ACC_W), jnp.float32)] * 2
            + [pltpu.SemaphoreType.DMA((4,)), pltpu.SemaphoreType.DMA((4,))]
        ),
    )
    return f(h, exrow, sdi)


def _tc_post(part, xt, h, a_s, a_d, pm, qm, em, gb, ln2w, ln2b):
    def body(p_ref, xt_ref, h_ref, as_ref, ad_ref, pm_ref, qm_ref, em_ref,
             gb_ref, g_ref, b_ref, out_ref):
        acc = p_ref[0] + p_ref[1]
        z = as_ref[...] + ad_ref[...]
        z = jnp.where(z >= 0.0, z, 0.2 * z)
        exs = jnp.exp(z)
        hv = h_ref[...]
        exw = jnp.dot(exs, em_ref[...], preferred_element_type=jnp.float32)
        num = jnp.dot(acc, pm_ref[...],
                      preferred_element_type=jnp.float32) + exw * hv
        den = jnp.dot(acc, qm_ref[...],
                      preferred_element_type=jnp.float32) + exw
        xg = num / (den + 1e-16) + gb_ref[...]
        y = xt_ref[...] + xg
        m = jnp.mean(y, axis=1, keepdims=True)
        yc = y - m
        v = jnp.mean(yc * yc, axis=1, keepdims=True)
        out_ref[...] = yc * lax.rsqrt(v + 1e-5) * g_ref[...] + b_ref[...]

    grid = (N // BLK,)
    return pl.pallas_call(
        body,
        grid=grid,
        in_specs=[
            pl.BlockSpec((NC, BLK, ACC_W), lambda i: (0, i, 0)),
            pl.BlockSpec((BLK, D), lambda i: (i, 0)),
            pl.BlockSpec((BLK, D), lambda i: (i, 0)),
            pl.BlockSpec((BLK, H), lambda i: (i, 0)),
            pl.BlockSpec((BLK, H), lambda i: (i, 0)),
            pl.BlockSpec((ACC_W, D), lambda i: (0, 0)),
            pl.BlockSpec((ACC_W, D), lambda i: (0, 0)),
            pl.BlockSpec((H, D), lambda i: (0, 0)),
            pl.BlockSpec((D,), lambda i: (0,)),
            pl.BlockSpec((D,), lambda i: (0,)),
            pl.BlockSpec((D,), lambda i: (0,)),
        ],
        out_specs=pl.BlockSpec((BLK, D), lambda i: (i, 0)),
        out_shape=jax.ShapeDtypeStruct((N, D), jnp.float32),
    )(part, xt, h, a_s, a_d, pm, qm, em, gb, ln2w, ln2b)


def kernel(x, edge_index, edge_attr, timestamps, ln1_w, ln1_b, W, att_src,
           att_dst, gat_bias, ln2_w, ln2_b):
    eyeH = jnp.eye(H, dtype=jnp.float32)
    am_s = (att_src[:, :, None] * eyeH[:, None, :]).reshape(D, H)
    am_d = (att_dst[:, :, None] * eyeH[:, None, :]).reshape(D, H)
    am = jnp.concatenate([am_s, am_d], axis=1)
    em = jnp.repeat(eyeH, C, axis=1).reshape(H, D)
    pm = jnp.concatenate(
        [jnp.eye(D, dtype=jnp.float32),
         jnp.zeros((ACC_W - D, D), jnp.float32)], axis=0)
    qm = jnp.concatenate(
        [jnp.zeros((D, D), jnp.float32), em,
         jnp.zeros((ACC_W - D - H, D), jnp.float32)], axis=0)

    xt, h, a_s, a_d = _tc_pre(x, W, am, ln1_w, ln1_b)

    src = jnp.concatenate([edge_index[0], jnp.zeros((EPAD - E,), jnp.int32)])
    dst = jnp.concatenate(
        [edge_index[1],
         N + jnp.arange(EPAD - E, dtype=jnp.int32) % (NACC - N)])
    zpad = jnp.zeros((NACC - N, 2 * H), jnp.float32)
    apair = jnp.concatenate(
        [jnp.concatenate([a_s, a_d], axis=1), zpad], axis=0)

    h_sw = h.reshape(N, H, 2, 16).transpose(0, 1, 3, 2).reshape(N, D)
    h_sw = h_sw.astype(jnp.bfloat16)

    exrow = _sc_ex_pass(apair, src, dst)
    sdi = jnp.stack([src.reshape(NW * NCH, CHUNK),
                     dst.reshape(NW * NCH, CHUNK)], axis=1)
    part = _sc_edge_pass(h_sw, exrow, sdi)

    return _tc_post(part, xt, h, a_s, a_d, pm, qm, em,
                    gat_bias, ln2_w, ln2_b)
